# baseline (speedup 1.0000x reference)
import functools

import jax
import jax.numpy as jnp
from jax.experimental import pallas as pl

_BLOCK = 2048


def _router_block(x_ref, w1_ref, b1_ref, w2_ref, b2_ref, wts_ref, logits_ref):
    h = jnp.dot(x_ref[...], w1_ref[...], preferred_element_type=jnp.float32)
    h = jnp.maximum(h + b1_ref[...], 0.0)
    logits = jnp.dot(h, w2_ref[...], preferred_element_type=jnp.float32)
    logits = logits + b2_ref[...]
    m = jnp.max(logits, axis=1, keepdims=True)
    e = jnp.exp(logits - m)
    wts_ref[...] = e / jnp.sum(e, axis=1, keepdims=True)
    logits_ref[...] = logits


@functools.partial(jax.jit, static_argnames=())
def kernel(x, W1, b1, W2, b2):
    n_tokens, feat_dim = x.shape
    hidden = W1.shape[1]
    n_experts = W2.shape[1]
    block = min(_BLOCK, n_tokens)
    grid = (n_tokens // block,)

    b1r = b1.reshape(1, hidden)
    b2r = b2.reshape(1, n_experts)

    wts, logits = pl.pallas_call(
        _router_block,
        grid=grid,
        in_specs=[
            pl.BlockSpec((block, feat_dim), lambda i: (i, 0)),
            pl.BlockSpec((feat_dim, hidden), lambda i: (0, 0)),
            pl.BlockSpec((1, hidden), lambda i: (0, 0)),
            pl.BlockSpec((hidden, n_experts), lambda i: (0, 0)),
            pl.BlockSpec((1, n_experts), lambda i: (0, 0)),
        ],
        out_specs=[
            pl.BlockSpec((block, n_experts), lambda i: (i, 0)),
            pl.BlockSpec((block, n_experts), lambda i: (i, 0)),
        ],
        out_shape=[
            jax.ShapeDtypeStruct((n_tokens, n_experts), jnp.float32),
            jax.ShapeDtypeStruct((n_tokens, n_experts), jnp.float32),
        ],
    )(x, W1, b1r, W2, b2r)
    return (wts, logits)


# speedup vs baseline: 1.0053x; 1.0053x over previous
import functools

import jax
import jax.numpy as jnp
from jax.experimental import pallas as pl

_BLOCK = 2048


def _router_block(x_ref, w1_ref, b1_ref, w2_ref, b2_ref, wts_ref, logits_ref):
    xb = x_ref[...].astype(jnp.bfloat16)
    w1 = w1_ref[...].astype(jnp.bfloat16)
    h = jnp.dot(xb, w1, preferred_element_type=jnp.float32)
    h = jnp.maximum(h + b1_ref[...], 0.0)
    logits = jnp.dot(h, w2_ref[...], preferred_element_type=jnp.float32)
    logits = logits + b2_ref[...]
    m = jnp.max(logits, axis=1, keepdims=True)
    e = jnp.exp(logits - m)
    wts_ref[...] = e / jnp.sum(e, axis=1, keepdims=True)
    logits_ref[...] = logits


@functools.partial(jax.jit, static_argnames=())
def kernel(x, W1, b1, W2, b2):
    n_tokens, feat_dim = x.shape
    hidden = W1.shape[1]
    n_experts = W2.shape[1]
    block = min(_BLOCK, n_tokens)
    grid = (n_tokens // block,)

    b1r = b1.reshape(1, hidden)
    b2r = b2.reshape(1, n_experts)

    wts, logits = pl.pallas_call(
        _router_block,
        grid=grid,
        in_specs=[
            pl.BlockSpec((block, feat_dim), lambda i: (i, 0)),
            pl.BlockSpec((feat_dim, hidden), lambda i: (0, 0)),
            pl.BlockSpec((1, hidden), lambda i: (0, 0)),
            pl.BlockSpec((hidden, n_experts), lambda i: (0, 0)),
            pl.BlockSpec((1, n_experts), lambda i: (0, 0)),
        ],
        out_specs=[
            pl.BlockSpec((block, n_experts), lambda i: (i, 0)),
            pl.BlockSpec((block, n_experts), lambda i: (i, 0)),
        ],
        out_shape=[
            jax.ShapeDtypeStruct((n_tokens, n_experts), jnp.float32),
            jax.ShapeDtypeStruct((n_tokens, n_experts), jnp.float32),
        ],
    )(x, W1, b1r, W2, b2r)
    return (wts, logits)
